# baseline (speedup 1.0000x reference)
import jax
import jax.numpy as jnp
from jax import lax
from jax.experimental import pallas as pl
from jax.experimental.pallas import tpu as pltpu


def _fused_body(x_ref, w_ref, b_ref, adj_ref, out_ref, sup_ref):
    @pl.when(pl.program_id(0) == 0)
    def _support():
        xb = x_ref[...].astype(jnp.bfloat16)
        wb = w_ref[...].astype(jnp.bfloat16)
        sup_ref[...] = lax.dot_general(
            xb, wb, (((1,), (0,)), ((), ())),
            preferred_element_type=jnp.float32).astype(jnp.bfloat16)

    a = adj_ref[...].astype(jnp.bfloat16)
    acc = lax.dot_general(
        a, sup_ref[...], (((1,), (0,)), ((), ())),
        preferred_element_type=jnp.float32)
    out_ref[...] = jnp.maximum(acc + b_ref[...], 0.0)


@jax.jit
def kernel(x, adj, W, b):
    M, K = adj.shape
    D_in = x.shape[1]
    D_out = W.shape[1]

    BM = 480 if M >= 480 else min(M, 256)
    nm = pl.cdiv(M, BM)

    out = pl.pallas_call(
        _fused_body,
        grid=(nm,),
        in_specs=[
            pl.BlockSpec((K, D_in), lambda i: (0, 0)),
            pl.BlockSpec((D_in, D_out), lambda i: (0, 0)),
            pl.BlockSpec((1, D_out), lambda i: (0, 0)),
            pl.BlockSpec((BM, K), lambda i: (i, 0)),
        ],
        out_specs=pl.BlockSpec((BM, D_out), lambda i: (i, 0)),
        out_shape=jax.ShapeDtypeStruct((M, D_out), jnp.float32),
        scratch_shapes=[pltpu.VMEM((K, D_out), jnp.bfloat16)],
        compiler_params=pltpu.CompilerParams(
            dimension_semantics=("arbitrary",)),
    )(x, W, b.reshape(1, D_out), adj)

    return out


# speedup vs baseline: 1.0066x; 1.0066x over previous
import jax
import jax.numpy as jnp
from jax import lax
from jax.experimental import pallas as pl
from jax.experimental.pallas import tpu as pltpu


def _fused_body(x_ref, w_ref, b_ref, adj_ref, out_ref, sup_ref):
    @pl.when(pl.program_id(0) == 0)
    def _support():
        xb = x_ref[...].astype(jnp.bfloat16)
        wb = w_ref[...].astype(jnp.bfloat16)
        sup_ref[...] = lax.dot_general(
            xb, wb, (((1,), (0,)), ((), ())),
            preferred_element_type=jnp.float32).astype(jnp.bfloat16)

    a = adj_ref[...].astype(jnp.bfloat16)
    acc = lax.dot_general(
        a, sup_ref[...], (((1,), (0,)), ((), ())),
        preferred_element_type=jnp.float32)
    out_ref[...] = jnp.maximum(acc + b_ref[...], 0.0)


@jax.jit
def kernel(x, adj, W, b):
    M, K = adj.shape
    D_in = x.shape[1]
    D_out = W.shape[1]

    BM = 400 if M % 400 == 0 else min(M, 256)
    nm = pl.cdiv(M, BM)

    out = pl.pallas_call(
        _fused_body,
        grid=(nm,),
        in_specs=[
            pl.BlockSpec((K, D_in), lambda i: (0, 0)),
            pl.BlockSpec((D_in, D_out), lambda i: (0, 0)),
            pl.BlockSpec((1, D_out), lambda i: (0, 0)),
            pl.BlockSpec((BM, K), lambda i: (i, 0)),
        ],
        out_specs=pl.BlockSpec((BM, D_out), lambda i: (i, 0)),
        out_shape=jax.ShapeDtypeStruct((M, D_out), jnp.float32),
        scratch_shapes=[pltpu.VMEM((K, D_out), jnp.bfloat16)],
        compiler_params=pltpu.CompilerParams(
            dimension_semantics=("arbitrary",)),
    )(x, W, b.reshape(1, D_out), adj)

    return out
